# baseline (speedup 1.0000x reference)
import functools

import jax
import jax.numpy as jnp
from jax import lax
from jax.experimental import pallas as pl
from jax.experimental.pallas import tpu as pltpu
from jax.experimental.pallas import tpu_sc as plsc

_N = 10000
_E = 320000
_NC = 2
_NS = 16
_CH = 80
_IB = 25
_D = 128
_DEGW = 16
_NP = 10240
_RPS = _NP // _NS

_R = 1000


def _make_segsum(split_edges: bool, with_deg: bool):
    mesh = plsc.VectorSubcoreMesh(core_axis_name="c", subcore_axis_name="s")
    epc = _E // _NC if split_edges else _E
    eps = epc // _NS
    nch = eps // _CH
    nb = nch // _IB
    assert eps % _CH == 0 and nch % _IB == 0 and _RPS % 8 == 0

    agg_t = jax.ShapeDtypeStruct((_NC, _NP, _D), jnp.float32)
    out_type = [agg_t, agg_t] if with_deg else agg_t
    scratch = [
        pltpu.VMEM((_IB, _CH), jnp.int32),
        pltpu.VMEM((_IB, _CH), jnp.int32),
        pltpu.VMEM((_CH, _D), jnp.float32),
        pltpu.VMEM((_CH, _D), jnp.float32),
        pltpu.VMEM_SHARED((_NP, _D), jnp.float32),
        pltpu.SemaphoreType.DMA,
        pltpu.SemaphoreType.DMA,
    ]

    def body(table, src_hbm, dst_hbm, zeros_d, *refs):
        if with_deg:
            (ones_d, agg_out, deg_out,
             src_vm, dst_vm, buf0, buf1, agg_sh, sem0, sem1) = refs
        else:
            (agg_out, src_vm, dst_vm, buf0, buf1, agg_sh, sem0, sem1) = refs
        c = lax.axis_index("c")
        s = lax.axis_index("s")
        r0 = s * _RPS
        w = c * _NS + s

        pltpu.sync_copy(zeros_d.at[pl.ds(r0, _RPS)],
                        agg_sh.at[pl.ds(r0, _RPS)])
        plsc.subcore_barrier()

        @pl.loop(0, nb)
        def _blk(bi):
            pltpu.sync_copy(src_hbm.at[w, bi], src_vm)
            pltpu.sync_copy(dst_hbm.at[w, bi], dst_vm)
            pltpu.async_copy(table.at[src_vm.at[0]], buf0, sem0)

            @pl.loop(0, _IB // 2)
            def _pair(i):
                a = 2 * i
                b = a + 1
                pltpu.async_copy(table.at[src_vm.at[b]], buf1, sem1)
                pltpu.make_async_copy(table.at[src_vm.at[a]], buf0,
                                      sem0).wait()
                pltpu.sync_copy(buf0, agg_sh.at[dst_vm.at[a]], add=True)

                @pl.when(b + 1 < _IB)
                def _next():
                    pltpu.async_copy(table.at[src_vm.at[b + 1]], buf0, sem0)

                pltpu.make_async_copy(table.at[src_vm.at[b]], buf1,
                                      sem1).wait()
                pltpu.sync_copy(buf1, agg_sh.at[dst_vm.at[b]], add=True)

            if _IB % 2 == 1:
                pltpu.make_async_copy(table.at[src_vm.at[_IB - 1]], buf0,
                                      sem0).wait()
                pltpu.sync_copy(buf0, agg_sh.at[dst_vm.at[_IB - 1]],
                                add=True)

        plsc.subcore_barrier()
        pltpu.sync_copy(agg_sh.at[pl.ds(r0, _RPS)],
                        agg_out.at[c, pl.ds(r0, _RPS)])

        if with_deg:
            plsc.subcore_barrier()
            pltpu.sync_copy(zeros_d.at[pl.ds(r0, _RPS)],
                            agg_sh.at[pl.ds(r0, _RPS)])
            pltpu.sync_copy(ones_d, buf0)
            plsc.subcore_barrier()

            @pl.loop(0, nb)
            def _dblk(bi):
                pltpu.sync_copy(dst_hbm.at[w, bi], dst_vm)
                for j in range(_IB):
                    pltpu.async_copy(buf0, agg_sh.at[dst_vm.at[j]],
                                     sem0, add=True)
                for j in range(_IB):
                    pltpu.make_async_copy(buf0, agg_sh.at[dst_vm.at[j]],
                                          sem0).wait()

            plsc.subcore_barrier()
            pltpu.sync_copy(agg_sh.at[pl.ds(r0, _RPS)],
                            deg_out.at[c, pl.ds(r0, _RPS)])

    return pl.kernel(body, out_type=out_type, mesh=mesh,
                     scratch_types=scratch)


_segsum_deg = _make_segsum(split_edges=True, with_deg=True)
_segsum = _make_segsum(split_edges=True, with_deg=False)
_segsum_cols = _make_segsum(split_edges=False, with_deg=False)


def _bdot(x, w):
    return jnp.dot(x.astype(jnp.bfloat16), w[...],
                   preferred_element_type=jnp.float32)


def _mm_body(x, w, o):
    o[...] = _bdot(x[...], w)


def _mm(x, w):
    n, din = x.shape
    dout = w.shape[1]
    return pl.pallas_call(
        _mm_body,
        grid=(n // _R,),
        in_specs=[pl.BlockSpec((_R, din), lambda i: (i, 0)),
                  pl.BlockSpec((din, dout), lambda i: (0, 0))],
        out_specs=pl.BlockSpec((_R, dout), lambda i: (i, 0)),
        out_shape=jax.ShapeDtypeStruct((n, dout), jnp.float32),
    )(x, w)


def _mm2c_body(xc, w, o):
    wv = w[...]
    o[...] = (jnp.dot(xc[0].astype(jnp.bfloat16), wv[:_D, :],
                      preferred_element_type=jnp.float32)
              + jnp.dot(xc[1].astype(jnp.bfloat16), wv[_D:, :],
                        preferred_element_type=jnp.float32))


def _mm2c(xc, w):
    dout = w.shape[1]
    return pl.pallas_call(
        _mm2c_body,
        grid=(_N // _R,),
        in_specs=[pl.BlockSpec((2, _R, _D), lambda i: (0, i, 0)),
                  pl.BlockSpec((2 * _D, dout), lambda i: (0, 0))],
        out_specs=pl.BlockSpec((_R, dout), lambda i: (i, 0)),
        out_shape=jax.ShapeDtypeStruct((_N, dout), jnp.float32),
    )(xc, w)


def _tc1_body(aggp, degp, r1, wl, b, h1c, inv):
    d = degp[0][:, :_DEGW] + degp[1][:, :_DEGW]
    iv = 1.0 / jnp.maximum(d, 1.0)
    inv[...] = iv
    mean = (aggp[0] + aggp[1]) * iv[:, :1]
    h = _bdot(mean, wl)
    h = jnp.maximum(h + b[...] + r1[...], 0.0)
    h1c[0] = h[:, :_D]
    h1c[1] = h[:, _D:]


def _tc1(aggp, degp, r1, wl, b):
    return pl.pallas_call(
        _tc1_body,
        grid=(_N // _R,),
        in_specs=[pl.BlockSpec((2, _R, _D), lambda i: (0, i, 0)),
                  pl.BlockSpec((2, _R, _D), lambda i: (0, i, 0)),
                  pl.BlockSpec((_R, 2 * _D), lambda i: (i, 0)),
                  pl.BlockSpec((_D, 2 * _D), lambda i: (0, 0)),
                  pl.BlockSpec((1, 2 * _D), lambda i: (0, 0))],
        out_specs=[pl.BlockSpec((2, _R, _D), lambda i: (0, i, 0)),
                   pl.BlockSpec((_R, _DEGW), lambda i: (i, 0))],
        out_shape=[jax.ShapeDtypeStruct((2, _N, _D), jnp.float32),
                   jax.ShapeDtypeStruct((_N, _DEGW), jnp.float32)],
    )(aggp, degp, r1, wl, b)


def _tc2_body(aggc, inv, r2, w2l, b2, w3l, h2, p3):
    iv = inv[:, :1]
    mcat = jnp.concatenate([aggc[0] * iv, aggc[1] * iv], axis=1)
    pre = _bdot(mcat, w2l)
    h = jnp.maximum(pre + b2[...] + r2[...], 0.0)
    h2[...] = h
    p3[...] = _bdot(h, w3l)


def _tc2(aggc, inv, r2, w2l, b2, w3l):
    return pl.pallas_call(
        _tc2_body,
        grid=(_N // _R,),
        in_specs=[pl.BlockSpec((2, _R, _D), lambda i: (0, i, 0)),
                  pl.BlockSpec((_R, _DEGW), lambda i: (i, 0)),
                  pl.BlockSpec((_R, 2 * _D), lambda i: (i, 0)),
                  pl.BlockSpec((2 * _D, 2 * _D), lambda i: (0, 0)),
                  pl.BlockSpec((1, 2 * _D), lambda i: (0, 0)),
                  pl.BlockSpec((2 * _D, _D), lambda i: (0, 0))],
        out_specs=[pl.BlockSpec((_R, 2 * _D), lambda i: (i, 0)),
                   pl.BlockSpec((_R, _D), lambda i: (i, 0))],
        out_shape=[jax.ShapeDtypeStruct((_N, 2 * _D), jnp.float32),
                   jax.ShapeDtypeStruct((_N, _D), jnp.float32)],
    )(aggc, inv, r2, w2l, b2, w3l)


def _tc3_body(aggp, inv, r3, b3, h3):
    mean = (aggp[0] + aggp[1]) * inv[:, :1]
    h3[...] = jnp.maximum(mean + b3[...] + r3[...], 0.0)


def _tc3(aggp, inv, r3, b3):
    return pl.pallas_call(
        _tc3_body,
        grid=(_N // _R,),
        in_specs=[pl.BlockSpec((2, _R, _D), lambda i: (0, i, 0)),
                  pl.BlockSpec((_R, _DEGW), lambda i: (i, 0)),
                  pl.BlockSpec((_R, _D), lambda i: (i, 0)),
                  pl.BlockSpec((1, _D), lambda i: (0, 0))],
        out_specs=pl.BlockSpec((_R, _D), lambda i: (i, 0)),
        out_shape=jax.ShapeDtypeStruct((_N, _D), jnp.float32),
    )(aggp, inv, r3, b3)


def _tc4_body(aggp, inv, r4, w4l, b4, o):
    mean = (aggp[0] + aggp[1]) * inv[:, :1]
    o[...] = _bdot(mean, w4l) + b4[...] + r4[...]


def _tc4(aggp, inv, r4, w4l, b4):
    return pl.pallas_call(
        _tc4_body,
        grid=(_N // _R,),
        in_specs=[pl.BlockSpec((2, _R, _D), lambda i: (0, i, 0)),
                  pl.BlockSpec((_R, _DEGW), lambda i: (i, 0)),
                  pl.BlockSpec((_R, _D), lambda i: (i, 0)),
                  pl.BlockSpec((_D, _D), lambda i: (0, 0)),
                  pl.BlockSpec((1, _D), lambda i: (0, 0))],
        out_specs=pl.BlockSpec((_R, _D), lambda i: (i, 0)),
        out_shape=jax.ShapeDtypeStruct((_N, _D), jnp.float32),
    )(aggp, inv, r4, w4l, b4)


def kernel(z, edge_index, W1l, b1, W1r, W2l, b2, W2r, W3l, b3, W3r, W4l, b4,
           W4r):
    f32 = jnp.float32
    src = edge_index[0].astype(jnp.int32)
    dst = edge_index[1].astype(jnp.int32)
    nw = _NC * _NS
    src_e = src.reshape(nw, -1, _IB, _CH)
    dst_e = dst.reshape(nw, -1, _IB, _CH)
    src_c = jnp.concatenate([src, src + _N]).reshape(nw, -1, _IB, _CH)
    dst_c = jnp.concatenate([dst, dst]).reshape(nw, -1, _IB, _CH)
    zeros_d = jnp.zeros((_NP, _D), f32)
    ones_d = jnp.ones((_CH, _D), f32)
    b1r, b2r, b3r, b4r = (b.reshape(1, -1) for b in (b1, b2, b3, b4))
    bf16 = jnp.bfloat16
    W1l, W1r, W2l, W2r, W3l, W3r, W4l, W4r = (
        w.astype(bf16) for w in (W1l, W1r, W2l, W2r, W3l, W3r, W4l, W4r))

    agg1, degp = _segsum_deg(z, src_e, dst_e, zeros_d, ones_d)
    r1 = _mm(z, W1r)
    h1c, inv = _tc1(agg1, degp, r1, W1l, b1r)

    agg2 = _segsum_cols(h1c.reshape(2 * _N, _D), src_c, dst_c, zeros_d)
    r2 = _mm2c(h1c, W2r)
    h2, p3 = _tc2(agg2, inv, r2, W2l, b2r, W3l)

    agg3 = _segsum(p3, src_e, dst_e, zeros_d)
    r3 = _mm(h2, W3r)
    h3 = _tc3(agg3, inv, r3, b3r)

    agg4 = _segsum(h3, src_e, dst_e, zeros_d)
    r4 = _mm(h3, W4r)
    return _tc4(agg4, inv, r4, W4l, b4r)


# speedup vs baseline: 1.1053x; 1.1053x over previous
import functools

import jax
import jax.numpy as jnp
from jax import lax
from jax.experimental import pallas as pl
from jax.experimental.pallas import tpu as pltpu
from jax.experimental.pallas import tpu_sc as plsc

_N = 10000
_E = 320000
_NC = 2
_NS = 16
_CH = 80
_IB = 25
_D = 128
_DEGW = 16
_NP = 10240
_RPS = _NP // _NS

_R = 1000


def _make_segsum(split_edges: bool, with_deg: bool):
    mesh = plsc.VectorSubcoreMesh(core_axis_name="c", subcore_axis_name="s")
    epc = _E // _NC if split_edges else _E
    eps = epc // _NS
    nch = eps // _CH
    nb = nch // _IB
    assert eps % _CH == 0 and nch % _IB == 0 and _RPS % 8 == 0

    agg_t = jax.ShapeDtypeStruct((_NC, _NP, _D), jnp.float32)
    out_type = [agg_t, agg_t] if with_deg else agg_t
    scratch = [
        pltpu.VMEM((_IB, _CH), jnp.int32),
        pltpu.VMEM((_IB, _CH), jnp.int32),
        pltpu.VMEM((_CH, _D), jnp.float32),
        pltpu.VMEM((_CH, _D), jnp.float32),
        pltpu.VMEM((_CH, _D), jnp.float32),
        pltpu.VMEM_SHARED((_NP, _D), jnp.float32),
        pltpu.SemaphoreType.DMA,
        pltpu.SemaphoreType.DMA,
        pltpu.SemaphoreType.DMA,
        pltpu.SemaphoreType.DMA,
        pltpu.SemaphoreType.DMA,
        pltpu.SemaphoreType.DMA,
    ]

    def body(table, src_hbm, dst_hbm, zeros_d, *refs):
        if with_deg:
            (ones_d, agg_out, deg_out, src_vm, dst_vm,
             buf0, buf1, buf2, agg_sh,
             gs0, gs1, gs2, ss0, ss1, ss2) = refs
        else:
            (agg_out, src_vm, dst_vm,
             buf0, buf1, buf2, agg_sh,
             gs0, gs1, gs2, ss0, ss1, ss2) = refs
        bufs = (buf0, buf1, buf2)
        gsems = (gs0, gs1, gs2)
        ssems = (ss0, ss1, ss2)

        def fire_g(k, r):
            pltpu.async_copy(table.at[src_vm.at[k]], bufs[r], gsems[r])

        def wait_g(k, r):
            pltpu.make_async_copy(table.at[src_vm.at[k]], bufs[r],
                                  gsems[r]).wait()

        def fire_s(k, r):
            pltpu.async_copy(bufs[r], agg_sh.at[dst_vm.at[k]], ssems[r],
                             add=True)

        def wait_s(k, r):
            pltpu.make_async_copy(bufs[r], agg_sh.at[dst_vm.at[k]],
                                  ssems[r]).wait()
        c = lax.axis_index("c")
        s = lax.axis_index("s")
        r0 = s * _RPS
        w = c * _NS + s

        pltpu.sync_copy(zeros_d.at[pl.ds(r0, _RPS)],
                        agg_sh.at[pl.ds(r0, _RPS)])
        plsc.subcore_barrier()

        assert _IB == 25
        @pl.loop(0, nb)
        def _blk(bi):
            pltpu.sync_copy(src_hbm.at[w, bi], src_vm)
            pltpu.sync_copy(dst_hbm.at[w, bi], dst_vm)
            fire_g(0, 0)

            @pl.loop(0, 8)
            def _triple(t):
                a = 3 * t

                @pl.when(t >= 1)
                def _w0():
                    wait_s(a - 2, 1)
                fire_g(a + 1, 1)
                wait_g(a, 0)
                fire_s(a, 0)

                @pl.when(t >= 1)
                def _w1():
                    wait_s(a - 1, 2)
                fire_g(a + 2, 2)
                wait_g(a + 1, 1)
                fire_s(a + 1, 1)

                wait_s(a, 0)
                fire_g(a + 3, 0)
                wait_g(a + 2, 2)
                fire_s(a + 2, 2)

            wait_g(24, 0)
            fire_s(24, 0)
            wait_s(22, 1)
            wait_s(23, 2)
            wait_s(24, 0)

        plsc.subcore_barrier()
        pltpu.sync_copy(agg_sh.at[pl.ds(r0, _RPS)],
                        agg_out.at[c, pl.ds(r0, _RPS)])

        if with_deg:
            plsc.subcore_barrier()
            pltpu.sync_copy(zeros_d.at[pl.ds(r0, _RPS)],
                            agg_sh.at[pl.ds(r0, _RPS)])
            pltpu.sync_copy(ones_d, buf0)
            plsc.subcore_barrier()

            @pl.loop(0, nb)
            def _dblk(bi):
                pltpu.sync_copy(dst_hbm.at[w, bi], dst_vm)
                for j in range(_IB):
                    pltpu.async_copy(buf0, agg_sh.at[dst_vm.at[j]],
                                     gs0, add=True)
                for j in range(_IB):
                    pltpu.make_async_copy(buf0, agg_sh.at[dst_vm.at[j]],
                                          gs0).wait()

            plsc.subcore_barrier()
            pltpu.sync_copy(agg_sh.at[pl.ds(r0, _RPS)],
                            deg_out.at[c, pl.ds(r0, _RPS)])

    return pl.kernel(body, out_type=out_type, mesh=mesh,
                     scratch_types=scratch)


_segsum_deg = _make_segsum(split_edges=True, with_deg=True)
_segsum = _make_segsum(split_edges=True, with_deg=False)
_segsum_cols = _make_segsum(split_edges=False, with_deg=False)


def _bdot(x, w):
    return jnp.dot(x.astype(jnp.bfloat16), w[...],
                   preferred_element_type=jnp.float32)


def _mm_body(x, w, o):
    o[...] = _bdot(x[...], w)


def _mm(x, w):
    n, din = x.shape
    dout = w.shape[1]
    return pl.pallas_call(
        _mm_body,
        grid=(n // _R,),
        in_specs=[pl.BlockSpec((_R, din), lambda i: (i, 0)),
                  pl.BlockSpec((din, dout), lambda i: (0, 0))],
        out_specs=pl.BlockSpec((_R, dout), lambda i: (i, 0)),
        out_shape=jax.ShapeDtypeStruct((n, dout), jnp.float32),
    )(x, w)


def _mm2c_body(xc, w, o):
    wv = w[...]
    o[...] = (jnp.dot(xc[0].astype(jnp.bfloat16), wv[:_D, :],
                      preferred_element_type=jnp.float32)
              + jnp.dot(xc[1].astype(jnp.bfloat16), wv[_D:, :],
                        preferred_element_type=jnp.float32))


def _mm2c(xc, w):
    dout = w.shape[1]
    return pl.pallas_call(
        _mm2c_body,
        grid=(_N // _R,),
        in_specs=[pl.BlockSpec((2, _R, _D), lambda i: (0, i, 0)),
                  pl.BlockSpec((2 * _D, dout), lambda i: (0, 0))],
        out_specs=pl.BlockSpec((_R, dout), lambda i: (i, 0)),
        out_shape=jax.ShapeDtypeStruct((_N, dout), jnp.float32),
    )(xc, w)


def _tc1_body(aggp, degp, r1, wl, b, h1c, inv):
    d = degp[0][:, :_DEGW] + degp[1][:, :_DEGW]
    iv = 1.0 / jnp.maximum(d, 1.0)
    inv[...] = iv
    mean = (aggp[0] + aggp[1]) * iv[:, :1]
    h = _bdot(mean, wl)
    h = jnp.maximum(h + b[...] + r1[...], 0.0)
    h1c[0] = h[:, :_D]
    h1c[1] = h[:, _D:]


def _tc1(aggp, degp, r1, wl, b):
    return pl.pallas_call(
        _tc1_body,
        grid=(_N // _R,),
        in_specs=[pl.BlockSpec((2, _R, _D), lambda i: (0, i, 0)),
                  pl.BlockSpec((2, _R, _D), lambda i: (0, i, 0)),
                  pl.BlockSpec((_R, 2 * _D), lambda i: (i, 0)),
                  pl.BlockSpec((_D, 2 * _D), lambda i: (0, 0)),
                  pl.BlockSpec((1, 2 * _D), lambda i: (0, 0))],
        out_specs=[pl.BlockSpec((2, _R, _D), lambda i: (0, i, 0)),
                   pl.BlockSpec((_R, _DEGW), lambda i: (i, 0))],
        out_shape=[jax.ShapeDtypeStruct((2, _N, _D), jnp.float32),
                   jax.ShapeDtypeStruct((_N, _DEGW), jnp.float32)],
    )(aggp, degp, r1, wl, b)


def _tc2_body(aggc, inv, r2, w2l, b2, w3l, h2, p3):
    iv = inv[:, :1]
    mcat = jnp.concatenate([aggc[0] * iv, aggc[1] * iv], axis=1)
    pre = _bdot(mcat, w2l)
    h = jnp.maximum(pre + b2[...] + r2[...], 0.0)
    h2[...] = h
    p3[...] = _bdot(h, w3l)


def _tc2(aggc, inv, r2, w2l, b2, w3l):
    return pl.pallas_call(
        _tc2_body,
        grid=(_N // _R,),
        in_specs=[pl.BlockSpec((2, _R, _D), lambda i: (0, i, 0)),
                  pl.BlockSpec((_R, _DEGW), lambda i: (i, 0)),
                  pl.BlockSpec((_R, 2 * _D), lambda i: (i, 0)),
                  pl.BlockSpec((2 * _D, 2 * _D), lambda i: (0, 0)),
                  pl.BlockSpec((1, 2 * _D), lambda i: (0, 0)),
                  pl.BlockSpec((2 * _D, _D), lambda i: (0, 0))],
        out_specs=[pl.BlockSpec((_R, 2 * _D), lambda i: (i, 0)),
                   pl.BlockSpec((_R, _D), lambda i: (i, 0))],
        out_shape=[jax.ShapeDtypeStruct((_N, 2 * _D), jnp.float32),
                   jax.ShapeDtypeStruct((_N, _D), jnp.float32)],
    )(aggc, inv, r2, w2l, b2, w3l)


def _tc3_body(aggp, inv, r3, b3, h3):
    mean = (aggp[0] + aggp[1]) * inv[:, :1]
    h3[...] = jnp.maximum(mean + b3[...] + r3[...], 0.0)


def _tc3(aggp, inv, r3, b3):
    return pl.pallas_call(
        _tc3_body,
        grid=(_N // _R,),
        in_specs=[pl.BlockSpec((2, _R, _D), lambda i: (0, i, 0)),
                  pl.BlockSpec((_R, _DEGW), lambda i: (i, 0)),
                  pl.BlockSpec((_R, _D), lambda i: (i, 0)),
                  pl.BlockSpec((1, _D), lambda i: (0, 0))],
        out_specs=pl.BlockSpec((_R, _D), lambda i: (i, 0)),
        out_shape=jax.ShapeDtypeStruct((_N, _D), jnp.float32),
    )(aggp, inv, r3, b3)


def _tc4_body(aggp, inv, r4, w4l, b4, o):
    mean = (aggp[0] + aggp[1]) * inv[:, :1]
    o[...] = _bdot(mean, w4l) + b4[...] + r4[...]


def _tc4(aggp, inv, r4, w4l, b4):
    return pl.pallas_call(
        _tc4_body,
        grid=(_N // _R,),
        in_specs=[pl.BlockSpec((2, _R, _D), lambda i: (0, i, 0)),
                  pl.BlockSpec((_R, _DEGW), lambda i: (i, 0)),
                  pl.BlockSpec((_R, _D), lambda i: (i, 0)),
                  pl.BlockSpec((_D, _D), lambda i: (0, 0)),
                  pl.BlockSpec((1, _D), lambda i: (0, 0))],
        out_specs=pl.BlockSpec((_R, _D), lambda i: (i, 0)),
        out_shape=jax.ShapeDtypeStruct((_N, _D), jnp.float32),
    )(aggp, inv, r4, w4l, b4)


def kernel(z, edge_index, W1l, b1, W1r, W2l, b2, W2r, W3l, b3, W3r, W4l, b4,
           W4r):
    f32 = jnp.float32
    src = edge_index[0].astype(jnp.int32)
    dst = edge_index[1].astype(jnp.int32)
    nw = _NC * _NS
    src_e = src.reshape(nw, -1, _IB, _CH)
    dst_e = dst.reshape(nw, -1, _IB, _CH)
    src_c = jnp.concatenate([src, src + _N]).reshape(nw, -1, _IB, _CH)
    dst_c = jnp.concatenate([dst, dst]).reshape(nw, -1, _IB, _CH)
    zeros_d = jnp.zeros((_NP, _D), f32)
    ones_d = jnp.ones((_CH, _D), f32)
    b1r, b2r, b3r, b4r = (b.reshape(1, -1) for b in (b1, b2, b3, b4))
    bf16 = jnp.bfloat16
    W1l, W1r, W2l, W2r, W3l, W3r, W4l, W4r = (
        w.astype(bf16) for w in (W1l, W1r, W2l, W2r, W3l, W3r, W4l, W4r))

    agg1, degp = _segsum_deg(z, src_e, dst_e, zeros_d, ones_d)
    r1 = _mm(z, W1r)
    h1c, inv = _tc1(agg1, degp, r1, W1l, b1r)

    agg2 = _segsum_cols(h1c.reshape(2 * _N, _D), src_c, dst_c, zeros_d)
    r2 = _mm2c(h1c, W2r)
    h2, p3 = _tc2(agg2, inv, r2, W2l, b2r, W3l)

    agg3 = _segsum(p3, src_e, dst_e, zeros_d)
    r3 = _mm(h2, W3r)
    h3 = _tc3(agg3, inv, r3, b3r)

    agg4 = _segsum(h3, src_e, dst_e, zeros_d)
    r4 = _mm(h3, W4r)
    return _tc4(agg4, inv, r4, W4l, b4r)
